# baseline (speedup 1.0000x reference)
import functools

import jax
import jax.numpy as jnp
from jax import lax
from jax.experimental import pallas as pl
from jax.experimental.pallas import tpu as pltpu
from jax.experimental.pallas import tpu_sc as plsc

N = 512
E1 = 512
E2 = 4096
PE = 16
EMB = 64

NC, NS, L = 2, 16, 16
NW = NC * NS
N1 = E1 // NW
N2 = E2 // NW
CT1 = E1 // 128
CT2 = E2 // 128

KOFF = [((k >> 3) << 21) + ((k & 7) << 7) for k in range(PE)]

_MESH = plsc.VectorSubcoreMesh(
    core_axis_name="c", subcore_axis_name="s", num_cores=NC, num_subcores=NS)


@functools.partial(
    pl.kernel,
    mesh=_MESH,
    compiler_params=pltpu.CompilerParams(use_tc_tiling_on_sc=False),
    out_type=(
        jax.ShapeDtypeStruct((2, CT1, 8, 128), jnp.float32),
        jax.ShapeDtypeStruct((2, CT2, 1024), jnp.float32),
    ),
    scratch_types=[
        pltpu.VMEM((N1,), jnp.int32),
        pltpu.VMEM((N1,), jnp.int32),
        pltpu.VMEM((2 * N1 * PE,), jnp.int32),
        pltpu.VMEM((2 * N1 * PE,), jnp.float32),
        pltpu.VMEM((PE, N1), jnp.float32),
        pltpu.VMEM((N2,), jnp.int32),
        pltpu.VMEM((N2,), jnp.int32),
        pltpu.VMEM((4 * N2,), jnp.int32),
        pltpu.VMEM((4 * N2,), jnp.int32),
        pltpu.VMEM((4 * N2,), jnp.int32),
        pltpu.VMEM((4 * N2,), jnp.int32),
        pltpu.VMEM((4 * N2,), jnp.float32),
        pltpu.VMEM((4 * N2,), jnp.float32),
        pltpu.VMEM((4 * N2,), jnp.float32),
        pltpu.VMEM((4 * N2,), jnp.float32),
        pltpu.SemaphoreType.DMA,
        pltpu.SemaphoreType.DMA,
        pltpu.SemaphoreType.DMA,
        pltpu.SemaphoreType.DMA,
        pltpu.SemaphoreType.DMA,
        pltpu.SemaphoreType.DMA,
    ],
)
def _sc_gather(ei1_hbm, tab1_hbm, ei2_hbm, tab2_hbm,
               x1_hbm, x2_hbm,
               r1_v, c1_v, i1_v, g1_v, avg_v,
               r2_v, c2_v, i2q0, i2q1, i2q2, i2q3,
               g2q0, g2q1, g2q2, g2q3,
               semi, sem1, s2q0, s2q1, s2q2, s2q3):
    i2_v = [i2q0, i2q1, i2q2, i2q3]
    g2_v = [g2q0, g2q1, g2q2, g2q3]
    sem2 = [s2q0, s2q1, s2q2, s2q3]
    wid = lax.axis_index("s") * NC + lax.axis_index("c")
    ct1 = wid // 8
    cl1 = (wid % 8) * N1

    ld = [
        pltpu.async_copy(ei2_hbm.at[pl.ds(wid * 256, N2)], r2_v, semi),
        pltpu.async_copy(ei2_hbm.at[pl.ds(wid * 256 + 128, N2)], c2_v, semi),
        pltpu.async_copy(ei1_hbm.at[pl.ds(ct1 * 256 + cl1, N1)], r1_v, semi),
        pltpu.async_copy(ei1_hbm.at[pl.ds(ct1 * 256 + 128 + cl1, N1)], c1_v, semi),
    ]
    ld[0].wait()
    ld[1].wait()

    cp2 = []
    for q in range(4):
        def i2_body(g, carry, q=q):
            sl = pl.ds(g * L, L)
            f = r2_v[sl] * N + c2_v[sl]
            base = ((f >> 7) << 10) + (f & 127)
            for k4 in range(4):
                i2_v[q][pl.ds((k4 * (N2 // L) + g) * L, L)] = (
                    base + KOFF[q * 4 + k4])
            return carry

        lax.fori_loop(0, N2 // L, i2_body, 0, unroll=False)
        cp2.append(pltpu.async_copy(tab2_hbm.at[i2_v[q]], g2_v[q], sem2[q]))

    ld[2].wait()
    ld[3].wait()
    r = r1_v[...]
    c = c1_v[...]
    f_rc = r * N + c
    f_cr = c * N + r
    brc = ((f_rc >> 7) << 10) + (f_rc & 127)
    bcr = ((f_cr >> 7) << 10) + (f_cr & 127)

    def i1_body(k, carry):
        koff_k = ((k >> 3) << 21) + ((k & 7) << 7)
        i1_v[pl.ds(k * N1, N1)] = brc + koff_k
        i1_v[pl.ds((PE + k) * N1, N1)] = bcr + koff_k
        return carry

    lax.fori_loop(0, PE, i1_body, 0, unroll=False)
    cp1 = pltpu.async_copy(tab1_hbm.at[i1_v], g1_v, sem1)

    cp1.wait()

    def avg_body(k, carry):
        row = (g1_v[pl.ds(k * N1, N1)] + g1_v[pl.ds((PE + k) * N1, N1)]) * 0.5
        avg_v[k, :] = row
        return carry

    lax.fori_loop(0, PE, avg_body, 0, unroll=False)
    wb = [
        pltpu.async_copy(avg_v.at[pl.ds(0, 8), :],
                         x1_hbm.at[0, ct1, :, pl.ds(cl1, N1)], semi),
        pltpu.async_copy(avg_v.at[pl.ds(8, 8), :],
                         x1_hbm.at[1, ct1, :, pl.ds(cl1, N1)], semi),
    ]
    for q in range(4):
        cp2[q].wait()
        h, off = q // 2, (q % 2) * 512
        wb.append(pltpu.async_copy(
            g2_v[q], x2_hbm.at[h, wid, pl.ds(off, 512)], semi))
    for w in wb:
        w.wait()


def _tc_body(x1_ref, w1_ref, b1_ref, x2_ref, w2_ref, b2_ref, o1_ref, o2_ref):
    dn = (((0,), (0,)), ((), ()))
    o1_ref[...] = (
        lax.dot_general(w1_ref[...], x1_ref[...], dn,
                        preferred_element_type=jnp.float32)
        + b1_ref[...]
    )
    o2_ref[...] = (
        lax.dot_general(w2_ref[...], x2_ref[...], dn,
                        preferred_element_type=jnp.float32)
        + b2_ref[...]
    )


_tc_encode = pl.pallas_call(
    _tc_body,
    out_shape=(
        jax.ShapeDtypeStruct((EMB, E1), jnp.float32),
        jax.ShapeDtypeStruct((EMB, E2), jnp.float32),
    ),
)


def _phys_view(tab):
    return (tab.T.reshape(PE // 8, 8, (N * N) // 128, 128)
            .transpose(0, 2, 1, 3).reshape(-1))


def _idx_view(ei):
    e = ei.shape[1]
    return ei.reshape(2, e // 128, 128).transpose(1, 0, 2).reshape(-1)


def _xt_view(stage, e):
    return (stage.reshape(2, e // 128, 8, 128)
            .transpose(0, 2, 1, 3).reshape(PE, e))


def kernel(edge_RWSE, batch, edge_index, e2e_edge_RWSE, e_batch, e2e_edge_index,
           W_enc, b_enc, W_e2e, b_e2e):
    del batch, e_batch
    s1, s2 = _sc_gather(
        _idx_view(edge_index), _phys_view(edge_RWSE),
        _idx_view(e2e_edge_index), _phys_view(e2e_edge_RWSE),
    )
    o1t, o2t = _tc_encode(
        _xt_view(s1, E1), W_enc, b_enc.reshape(EMB, 1),
        _xt_view(s2, E2), W_e2e, b_e2e.reshape(EMB, 1),
    )
    return o1t.T, o2t.T


# speedup vs baseline: 1.0129x; 1.0129x over previous
import functools

import jax
import jax.numpy as jnp
from jax import lax
from jax.experimental import pallas as pl
from jax.experimental.pallas import tpu as pltpu
from jax.experimental.pallas import tpu_sc as plsc

N = 512
E1 = 512
E2 = 4096
PE = 16
EMB = 64

NC, NS, L = 2, 16, 16
NW = NC * NS
N1 = E1 // NW
N2 = E2 // NW
CT1 = E1 // 128
CT2 = E2 // 128

KOFF = [((k >> 3) << 21) + ((k & 7) << 7) for k in range(PE)]

_MESH = plsc.VectorSubcoreMesh(
    core_axis_name="c", subcore_axis_name="s", num_cores=NC, num_subcores=NS)


@functools.partial(
    pl.kernel,
    mesh=_MESH,
    compiler_params=pltpu.CompilerParams(use_tc_tiling_on_sc=False),
    out_type=(
        jax.ShapeDtypeStruct((2, CT1, 8, 128), jnp.float32),
        jax.ShapeDtypeStruct((2, CT2, 1024), jnp.float32),
    ),
    scratch_types=[
        pltpu.VMEM((N1,), jnp.int32),
        pltpu.VMEM((N1,), jnp.int32),
        pltpu.VMEM((2 * N1 * PE,), jnp.int32),
        pltpu.VMEM((2 * N1 * PE,), jnp.float32),
        pltpu.VMEM((PE, N1), jnp.float32),
        pltpu.VMEM((N2,), jnp.int32),
        pltpu.VMEM((N2,), jnp.int32),
        pltpu.VMEM((4 * N2,), jnp.int32),
        pltpu.VMEM((4 * N2,), jnp.int32),
        pltpu.VMEM((4 * N2,), jnp.int32),
        pltpu.VMEM((4 * N2,), jnp.int32),
        pltpu.VMEM((4 * N2,), jnp.float32),
        pltpu.VMEM((4 * N2,), jnp.float32),
        pltpu.VMEM((4 * N2,), jnp.float32),
        pltpu.VMEM((4 * N2,), jnp.float32),
        pltpu.SemaphoreType.DMA,
        pltpu.SemaphoreType.DMA,
        pltpu.SemaphoreType.DMA,
        pltpu.SemaphoreType.DMA,
        pltpu.SemaphoreType.DMA,
        pltpu.SemaphoreType.DMA,
    ],
)
def _sc_gather(ei1_hbm, tab1_hbm, ei2_hbm, tab2_hbm,
               x1_hbm, x2_hbm,
               r1_v, c1_v, i1_v, g1_v, avg_v,
               r2_v, c2_v, i2q0, i2q1, i2q2, i2q3,
               g2q0, g2q1, g2q2, g2q3,
               semi, sem1, s2q0, s2q1, s2q2, s2q3):
    i2_v = [i2q0, i2q1, i2q2, i2q3]
    g2_v = [g2q0, g2q1, g2q2, g2q3]
    sem2 = [s2q0, s2q1, s2q2, s2q3]
    wid = lax.axis_index("s") * NC + lax.axis_index("c")
    ct1 = wid // 8
    cl1 = (wid % 8) * N1

    ld = [
        pltpu.async_copy(ei1_hbm.at[pl.ds(ct1 * 256 + cl1, N1)], r1_v, semi),
        pltpu.async_copy(ei1_hbm.at[pl.ds(ct1 * 256 + 128 + cl1, N1)], c1_v, semi),
        pltpu.async_copy(ei2_hbm.at[pl.ds(wid * 256, N2)], r2_v, semi),
        pltpu.async_copy(ei2_hbm.at[pl.ds(wid * 256 + 128, N2)], c2_v, semi),
    ]
    ld[0].wait()
    ld[1].wait()

    r = r1_v[...]
    c = c1_v[...]
    f_rc = r * N + c
    f_cr = c * N + r
    brc = ((f_rc >> 7) << 10) + (f_rc & 127)
    bcr = ((f_cr >> 7) << 10) + (f_cr & 127)

    def i1_body(k, carry):
        koff_k = ((k >> 3) << 21) + ((k & 7) << 7)
        i1_v[pl.ds(k * N1, N1)] = brc + koff_k
        i1_v[pl.ds((PE + k) * N1, N1)] = bcr + koff_k
        return carry

    lax.fori_loop(0, PE, i1_body, 0, unroll=False)
    cp1 = pltpu.async_copy(tab1_hbm.at[i1_v], g1_v, sem1)

    ld[2].wait()
    ld[3].wait()

    def i2_body(g, carry):
        sl = pl.ds(g * L, L)
        f = r2_v[sl] * N + c2_v[sl]
        base = ((f >> 7) << 10) + (f & 127)
        for q in range(4):
            for k4 in range(4):
                i2_v[q][pl.ds((k4 * (N2 // L) + g) * L, L)] = (
                    base + KOFF[q * 4 + k4])
        return carry

    lax.fori_loop(0, N2 // L, i2_body, 0, unroll=False)
    cp2 = [pltpu.async_copy(tab2_hbm.at[i2_v[q]], g2_v[q], sem2[q])
           for q in range(4)]

    cp1.wait()

    def avg_body(k, carry):
        row = (g1_v[pl.ds(k * N1, N1)] + g1_v[pl.ds((PE + k) * N1, N1)]) * 0.5
        avg_v[k, :] = row
        return carry

    lax.fori_loop(0, PE, avg_body, 0, unroll=False)
    wb = [
        pltpu.async_copy(avg_v.at[pl.ds(0, 8), :],
                         x1_hbm.at[0, ct1, :, pl.ds(cl1, N1)], semi),
        pltpu.async_copy(avg_v.at[pl.ds(8, 8), :],
                         x1_hbm.at[1, ct1, :, pl.ds(cl1, N1)], semi),
    ]
    for q in range(4):
        cp2[q].wait()
        h, off = q // 2, (q % 2) * 512
        wb.append(pltpu.async_copy(
            g2_v[q], x2_hbm.at[h, wid, pl.ds(off, 512)], semi))
    for w in wb:
        w.wait()


def _tc_body(x1_ref, w1_ref, b1_ref, x2_ref, w2_ref, b2_ref, o1_ref, o2_ref):
    dn = (((0,), (0,)), ((), ()))
    o1_ref[...] = (
        lax.dot_general(w1_ref[...], x1_ref[...], dn,
                        preferred_element_type=jnp.float32)
        + b1_ref[...]
    )
    o2_ref[...] = (
        lax.dot_general(w2_ref[...], x2_ref[...], dn,
                        preferred_element_type=jnp.float32)
        + b2_ref[...]
    )


_tc_encode = pl.pallas_call(
    _tc_body,
    out_shape=(
        jax.ShapeDtypeStruct((EMB, E1), jnp.float32),
        jax.ShapeDtypeStruct((EMB, E2), jnp.float32),
    ),
)


def _phys_view(tab):
    return (tab.T.reshape(PE // 8, 8, (N * N) // 128, 128)
            .transpose(0, 2, 1, 3).reshape(-1))


def _idx_view(ei):
    e = ei.shape[1]
    return ei.reshape(2, e // 128, 128).transpose(1, 0, 2).reshape(-1)


def _xt_view(stage, e):
    return (stage.reshape(2, e // 128, 8, 128)
            .transpose(0, 2, 1, 3).reshape(PE, e))


def kernel(edge_RWSE, batch, edge_index, e2e_edge_RWSE, e_batch, e2e_edge_index,
           W_enc, b_enc, W_e2e, b_e2e):
    del batch, e_batch
    s1, s2 = _sc_gather(
        _idx_view(edge_index), _phys_view(edge_RWSE),
        _idx_view(e2e_edge_index), _phys_view(e2e_edge_RWSE),
    )
    o1t, o2t = _tc_encode(
        _xt_view(s1, E1), W_enc, b_enc.reshape(EMB, 1),
        _xt_view(s2, E2), W_e2e, b_e2e.reshape(EMB, 1),
    )
    return o1t.T, o2t.T
